# baseline (speedup 1.0000x reference)
import jax
import jax.numpy as jnp
from jax import lax
from jax.experimental import pallas as pl
from jax.experimental.pallas import tpu as pltpu
from jax.experimental.pallas import tpu_sc as plsc

N = 10000
D = 128
E = 320000
K = 320
NCHUNK = E // K
NC = 2
NS = 16
NW = NC * NS
ITERS = -(-NCHUNK // NS)
NPAD = 10240
RZ = NPAD // NS

_MESH = plsc.VectorSubcoreMesh(core_axis_name="c", subcore_axis_name="s")


def _make_agg_body(width):
    def _sc_agg_body(h_hbm, src_hbm, dst_hbm, z_hbm, out_hbm,
                     src_v, dst_v, rows_v, acc, sem):
        cid = lax.axis_index("c")
        sid = lax.axis_index("s")
        r0 = sid * RZ
        @pl.when(cid == 0)
        def _():
            for j in range(RZ // K):
                pltpu.sync_copy(z_hbm, acc.at[pl.ds(r0 + j * K, K)])

        plsc.subcore_barrier()

        def it(i, carry):
            c = sid + NS * i
            live = (cid == 0) & (c < NCHUNK)

            @pl.when(live)
            def _():
                pltpu.sync_copy(src_hbm.at[pl.ds(c * K, K)], src_v)
                pltpu.sync_copy(dst_hbm.at[pl.ds(c * K, K)], dst_v)
                pltpu.async_copy(h_hbm.at[src_v], rows_v, sem).wait()

            for t in range(NS):
                plsc.subcore_barrier()

                @pl.when(live & (sid == t))
                def _():
                    pltpu.sync_copy(rows_v, acc.at[dst_v], add=True)

            return carry

        lax.fori_loop(0, ITERS, it, 0)
        plsc.subcore_barrier()

        @pl.when(cid == 0)
        def _():
            pltpu.sync_copy(acc.at[pl.ds(r0, RZ)], out_hbm.at[pl.ds(r0, RZ)])


    return _sc_agg_body


def _make_agg(width):
    return pl.kernel(
        _make_agg_body(width),
        out_type=jax.ShapeDtypeStruct((NPAD, width), jnp.float32),
        mesh=_MESH,
        scratch_types=[
            pltpu.VMEM((K,), jnp.int32),
            pltpu.VMEM((K,), jnp.int32),
            pltpu.VMEM((K, width), jnp.float32),
            pltpu.VMEM_SHARED((NPAD, width), jnp.float32),
            pltpu.SemaphoreType.DMA,
        ],
    )


DEGW = 128
_sc_agg = _make_agg(D)
_sc_deg = _make_agg(DEGW)


def _tc_pre_body(x_ref, w_ref, aggd_ref, h1p_ref, dis_ref):
    deg = aggd_ref[...][:N, 0:1] + 1.0
    dis = lax.rsqrt(deg)
    h = jnp.dot(x_ref[...], w_ref[...], preferred_element_type=jnp.float32)
    h1p_ref[...] = h * dis
    dis_ref[...] = dis


_tc_pre = pl.pallas_call(
    _tc_pre_body,
    out_shape=(
        jax.ShapeDtypeStruct((N, D), jnp.float32),
        jax.ShapeDtypeStruct((N, 1), jnp.float32),
    ),
)


def _tc_mid_body(p_ref, h1p_ref, dis_ref, b1_ref, w2_ref, h2p_ref):
    dis = dis_ref[...]
    a = dis * (p_ref[...][:N] + h1p_ref[...]) + b1_ref[...]
    z = jnp.maximum(a, 0.0)
    h2p_ref[...] = dis * jnp.dot(z, w2_ref[...],
                                 preferred_element_type=jnp.float32)


_tc_mid = pl.pallas_call(
    _tc_mid_body,
    out_shape=jax.ShapeDtypeStruct((N, D), jnp.float32),
)


def _tc_post_body(p_ref, h2p_ref, dis_ref, b2_ref, out_ref):
    out_ref[...] = dis_ref[...] * (p_ref[...][:N] + h2p_ref[...]) + b2_ref[...]


_tc_post = pl.pallas_call(
    _tc_post_body,
    out_shape=jax.ShapeDtypeStruct((N, D), jnp.float32),
)


def kernel(x, edge_index, W1, b1, W2, b2):
    src = edge_index[0]
    dst = edge_index[1]
    z128 = jnp.zeros((K, D), jnp.float32)
    z32 = jnp.zeros((K, DEGW), jnp.float32)
    ones_n = jnp.ones((N, DEGW), jnp.float32)

    aggd = _sc_deg(ones_n, src, dst, z32)
    h1p, dis = _tc_pre(x, W1, aggd)
    p1 = _sc_agg(h1p, src, dst, z128)
    h2p = _tc_mid(p1, h1p, dis, b1.reshape(1, D), W2)
    p2 = _sc_agg(h2p, src, dst, z128)
    return _tc_post(p2, h2p, dis, b2.reshape(1, D))


# speedup vs baseline: 1.0323x; 1.0323x over previous
import jax
import jax.numpy as jnp
from jax import lax
from jax.experimental import pallas as pl
from jax.experimental.pallas import tpu as pltpu
from jax.experimental.pallas import tpu_sc as plsc

N = 10000
D = 128
E = 320000
K = 320
NCHUNK = E // K
NC = 2
NS = 16
NW = NC * NS
ITERS = -(-NCHUNK // NS)
NPAD = 10240
RZ = NPAD // NS

_MESH = plsc.VectorSubcoreMesh(core_axis_name="c", subcore_axis_name="s")


def _make_agg_body(width):
    def _sc_agg_body(h_hbm, src_hbm, dst_hbm, z_hbm, out_hbm,
                     src_v, dst_v, rows_v, acc, sem):
        cid = lax.axis_index("c")
        sid = lax.axis_index("s")
        r0 = sid * RZ
        @pl.when(cid == 0)
        def _():
            for j in range(RZ // K):
                pltpu.sync_copy(z_hbm, acc.at[pl.ds(r0 + j * K, K)])

        plsc.subcore_barrier()

        def it(i, carry):
            c = sid + NS * i
            live = (cid == 0) & (c < NCHUNK)

            @pl.when(live)
            def _():
                pltpu.sync_copy(src_hbm.at[pl.ds(c * K, K)], src_v)
                pltpu.sync_copy(dst_hbm.at[pl.ds(c * K, K)], dst_v)
                pltpu.async_copy(h_hbm.at[src_v], rows_v, sem).wait()

            for t in range(NS):
                plsc.subcore_barrier()

                @pl.when(live & (sid == t))
                def _():
                    pltpu.sync_copy(rows_v, acc.at[dst_v], add=True)

            return carry

        lax.fori_loop(0, ITERS, it, 0)
        plsc.subcore_barrier()

        @pl.when(cid == 0)
        def _():
            pltpu.sync_copy(acc.at[pl.ds(r0, RZ)], out_hbm.at[pl.ds(r0, RZ)])


    return _sc_agg_body


def _make_agg(width):
    return pl.kernel(
        _make_agg_body(width),
        out_type=jax.ShapeDtypeStruct((NPAD, width), jnp.float32),
        mesh=_MESH,
        scratch_types=[
            pltpu.VMEM((K,), jnp.int32),
            pltpu.VMEM((K,), jnp.int32),
            pltpu.VMEM((K, width), jnp.float32),
            pltpu.VMEM_SHARED((NPAD, width), jnp.float32),
            pltpu.SemaphoreType.DMA,
        ],
    )


def _make_deg_body(width):
    def _sc_deg_body(ones_hbm, dst_hbm, z_hbm, out_hbm,
                     dst_v, rows_v, acc, sem):
        cid = lax.axis_index("c")
        sid = lax.axis_index("s")
        r0 = sid * RZ

        @pl.when(cid == 0)
        def _():
            pltpu.sync_copy(ones_hbm, rows_v)
            for j in range(RZ // K):
                pltpu.sync_copy(z_hbm, acc.at[pl.ds(r0 + j * K, K)])

        plsc.subcore_barrier()

        def it(i, carry):
            c = sid + NS * i
            live = (cid == 0) & (c < NCHUNK)

            @pl.when(live)
            def _():
                pltpu.sync_copy(dst_hbm.at[pl.ds(c * K, K)], dst_v)

            for t in range(NS):
                plsc.subcore_barrier()

                @pl.when(live & (sid == t))
                def _():
                    pltpu.sync_copy(rows_v, acc.at[dst_v], add=True)

            return carry

        lax.fori_loop(0, ITERS, it, 0)
        plsc.subcore_barrier()

        @pl.when(cid == 0)
        def _():
            pltpu.sync_copy(acc.at[pl.ds(r0, RZ)], out_hbm.at[pl.ds(r0, RZ)])

    return _sc_deg_body


def _make_deg(width):
    return pl.kernel(
        _make_deg_body(width),
        out_type=jax.ShapeDtypeStruct((NPAD, width), jnp.float32),
        mesh=_MESH,
        scratch_types=[
            pltpu.VMEM((K,), jnp.int32),
            pltpu.VMEM((K, width), jnp.float32),
            pltpu.VMEM_SHARED((NPAD, width), jnp.float32),
            pltpu.SemaphoreType.DMA,
        ],
    )


DEGW = 128
_sc_agg = _make_agg(D)
_sc_deg = _make_deg(DEGW)


def _tc_pre_body(x_ref, w_ref, aggd_ref, h1p_ref, dis_ref):
    deg = aggd_ref[...][:N, 0:1] + 1.0
    dis = lax.rsqrt(deg)
    h = jnp.dot(x_ref[...], w_ref[...], preferred_element_type=jnp.float32)
    h1p_ref[...] = h * dis
    dis_ref[...] = dis


_tc_pre = pl.pallas_call(
    _tc_pre_body,
    out_shape=(
        jax.ShapeDtypeStruct((N, D), jnp.float32),
        jax.ShapeDtypeStruct((N, 1), jnp.float32),
    ),
)


def _tc_mid_body(p_ref, h1p_ref, dis_ref, b1_ref, w2_ref, h2p_ref):
    dis = dis_ref[...]
    a = dis * (p_ref[...][:N] + h1p_ref[...]) + b1_ref[...]
    z = jnp.maximum(a, 0.0)
    h2p_ref[...] = dis * jnp.dot(z, w2_ref[...],
                                 preferred_element_type=jnp.float32)


_tc_mid = pl.pallas_call(
    _tc_mid_body,
    out_shape=jax.ShapeDtypeStruct((N, D), jnp.float32),
)


def _tc_post_body(p_ref, h2p_ref, dis_ref, b2_ref, out_ref):
    out_ref[...] = dis_ref[...] * (p_ref[...][:N] + h2p_ref[...]) + b2_ref[...]


_tc_post = pl.pallas_call(
    _tc_post_body,
    out_shape=jax.ShapeDtypeStruct((N, D), jnp.float32),
)


def kernel(x, edge_index, W1, b1, W2, b2):
    src = edge_index[0]
    dst = edge_index[1]
    z128 = jnp.zeros((K, D), jnp.float32)
    z32 = jnp.zeros((K, DEGW), jnp.float32)
    ones_k = jnp.ones((K, DEGW), jnp.float32)

    aggd = _sc_deg(ones_k, dst, z32)
    h1p, dis = _tc_pre(x, W1, aggd)
    p1 = _sc_agg(h1p, src, dst, z128)
    h2p = _tc_mid(p1, h1p, dis, b1.reshape(1, D), W2)
    p2 = _sc_agg(h2p, src, dst, z128)
    return _tc_post(p2, h2p, dis, b2.reshape(1, D))


# speedup vs baseline: 1.8101x; 1.7534x over previous
import jax
import jax.numpy as jnp
from jax import lax
from jax.experimental import pallas as pl
from jax.experimental.pallas import tpu as pltpu
from jax.experimental.pallas import tpu_sc as plsc

N = 10000
D = 128
E = 320000
K = 160
NCHUNK = E // K
NC = 2
NS = 16
NW = NC * NS
ITERS = -(-NCHUNK // NW)
NPAD = 10240
RZ = NPAD // NS

_MESH = plsc.VectorSubcoreMesh(core_axis_name="c", subcore_axis_name="s")


def _sc_agg_body(h_hbm, src_hbm, dst_hbm, z_hbm, out_hbm,
                 srcA, dstA, rowsA, srcB, dstB, rowsB, acc, semA, semB):
    cid = lax.axis_index("c")
    sid = lax.axis_index("s")
    r0 = sid * RZ
    for j in range(RZ // K):
        pltpu.sync_copy(z_hbm, acc.at[pl.ds(r0 + j * K, K)])
    plsc.subcore_barrier()

    def it(i, carry):
        c = (sid * NC + cid) + NW * i
        live = c < NCHUNK

        @pl.when(live & (cid == 0))
        def _():
            pltpu.sync_copy(src_hbm.at[pl.ds(c * K, K)], srcA)
            pltpu.sync_copy(dst_hbm.at[pl.ds(c * K, K)], dstA)
            pltpu.async_copy(h_hbm.at[srcA], rowsA, semA).wait()

        @pl.when(live & (cid == 1))
        def _():
            pltpu.sync_copy(src_hbm.at[pl.ds(c * K, K)], srcB)
            pltpu.sync_copy(dst_hbm.at[pl.ds(c * K, K)], dstB)
            pltpu.async_copy(h_hbm.at[srcB], rowsB, semB).wait()

        for t in range(NS):
            plsc.subcore_barrier()

            @pl.when(live & (sid == t) & (cid == 0))
            def _():
                pltpu.sync_copy(rowsA, acc.at[dstA], add=True)

            @pl.when(live & (sid == t) & (cid == 1))
            def _():
                pltpu.sync_copy(rowsB, acc.at[dstB], add=True)

        return carry

    lax.fori_loop(0, ITERS, it, 0)
    plsc.subcore_barrier()
    pltpu.sync_copy(acc.at[pl.ds(r0, RZ)],
                    out_hbm.at[pl.ds(cid * NPAD + r0, RZ)])


_sc_agg = pl.kernel(
    _sc_agg_body,
    out_type=jax.ShapeDtypeStruct((NC * NPAD, D), jnp.float32),
    mesh=_MESH,
    scratch_types=[
        pltpu.VMEM((K,), jnp.int32),
        pltpu.VMEM((K,), jnp.int32),
        pltpu.VMEM((K, D), jnp.float32),
        pltpu.VMEM((K,), jnp.int32),
        pltpu.VMEM((K,), jnp.int32),
        pltpu.VMEM((K, D), jnp.float32),
        pltpu.VMEM_SHARED((NPAD, D), jnp.float32),
        pltpu.SemaphoreType.DMA,
        pltpu.SemaphoreType.DMA,
    ],
)


def _sc_deg_body(ones_hbm, dst_hbm, z_hbm, out_hbm,
                 dstA, dstB, rows_v, acc, sem):
    cid = lax.axis_index("c")
    sid = lax.axis_index("s")
    r0 = sid * RZ
    pltpu.sync_copy(ones_hbm, rows_v)
    for j in range(RZ // K):
        pltpu.sync_copy(z_hbm, acc.at[pl.ds(r0 + j * K, K)])
    plsc.subcore_barrier()

    def it(i, carry):
        c = (sid * NC + cid) + NW * i
        live = c < NCHUNK

        @pl.when(live & (cid == 0))
        def _():
            pltpu.sync_copy(dst_hbm.at[pl.ds(c * K, K)], dstA)

        @pl.when(live & (cid == 1))
        def _():
            pltpu.sync_copy(dst_hbm.at[pl.ds(c * K, K)], dstB)

        for t in range(NS):
            plsc.subcore_barrier()

            @pl.when(live & (sid == t) & (cid == 0))
            def _():
                pltpu.sync_copy(rows_v, acc.at[dstA], add=True)

            @pl.when(live & (sid == t) & (cid == 1))
            def _():
                pltpu.sync_copy(rows_v, acc.at[dstB], add=True)

        return carry

    lax.fori_loop(0, ITERS, it, 0)
    plsc.subcore_barrier()
    pltpu.sync_copy(acc.at[pl.ds(r0, RZ)],
                    out_hbm.at[pl.ds(cid * NPAD + r0, RZ)])


_sc_deg = pl.kernel(
    _sc_deg_body,
    out_type=jax.ShapeDtypeStruct((NC * NPAD, D), jnp.float32),
    mesh=_MESH,
    scratch_types=[
        pltpu.VMEM((K,), jnp.int32),
        pltpu.VMEM((K,), jnp.int32),
        pltpu.VMEM((K, D), jnp.float32),
        pltpu.VMEM_SHARED((NPAD, D), jnp.float32),
        pltpu.SemaphoreType.DMA,
    ],
)


def _tc_pre_body(x_ref, w_ref, aggd_ref, h1p_ref, dis_ref):
    a = aggd_ref[...]
    deg = a[:N, 0:1] + a[NPAD:NPAD + N, 0:1] + 1.0
    dis = lax.rsqrt(deg)
    h = jnp.dot(x_ref[...], w_ref[...], preferred_element_type=jnp.float32)
    h1p_ref[...] = h * dis
    dis_ref[...] = dis


_tc_pre = pl.pallas_call(
    _tc_pre_body,
    out_shape=(
        jax.ShapeDtypeStruct((N, D), jnp.float32),
        jax.ShapeDtypeStruct((N, 1), jnp.float32),
    ),
)


def _tc_mid_body(p_ref, h1p_ref, dis_ref, b1_ref, w2_ref, h2p_ref):
    p = p_ref[...]
    dis = dis_ref[...]
    a = dis * (p[:N] + p[NPAD:NPAD + N] + h1p_ref[...]) + b1_ref[...]
    z = jnp.maximum(a, 0.0)
    h2p_ref[...] = dis * jnp.dot(z, w2_ref[...],
                                 preferred_element_type=jnp.float32)


_tc_mid = pl.pallas_call(
    _tc_mid_body,
    out_shape=jax.ShapeDtypeStruct((N, D), jnp.float32),
)


def _tc_post_body(p_ref, h2p_ref, dis_ref, b2_ref, out_ref):
    p = p_ref[...]
    out_ref[...] = dis_ref[...] * (p[:N] + p[NPAD:NPAD + N]
                                   + h2p_ref[...]) + b2_ref[...]


_tc_post = pl.pallas_call(
    _tc_post_body,
    out_shape=jax.ShapeDtypeStruct((N, D), jnp.float32),
)


def kernel(x, edge_index, W1, b1, W2, b2):
    src = edge_index[0]
    dst = edge_index[1]
    z128 = jnp.zeros((K, D), jnp.float32)
    ones_k = jnp.ones((K, D), jnp.float32)

    aggd = _sc_deg(ones_k, dst, z128)
    h1p, dis = _tc_pre(x, W1, aggd)
    p1 = _sc_agg(h1p, src, dst, z128)
    h2p = _tc_mid(p1, h1p, dis, b1.reshape(1, D), W2)
    p2 = _sc_agg(h2p, src, dst, z128)
    return _tc_post(p2, h2p, dis, b2.reshape(1, D))


# speedup vs baseline: 1.8516x; 1.0229x over previous
import jax
import jax.numpy as jnp
from jax import lax
from jax.experimental import pallas as pl
from jax.experimental.pallas import tpu as pltpu
from jax.experimental.pallas import tpu_sc as plsc

N = 10000
D = 128
E = 320000
K = 160
NCHUNK = E // K
NC = 2
NS = 16
NW = NC * NS
ITERS = -(-NCHUNK // NW)
NPAD = 10240
RZ = NPAD // NS
KD = 320
NCHUNKD = E // KD
ITERSD = -(-NCHUNKD // NW)

_MESH = plsc.VectorSubcoreMesh(core_axis_name="c", subcore_axis_name="s")


def _sc_agg_body(h_hbm, src_hbm, dst_hbm, z_hbm, out_hbm,
                 srcA, dstA, rowsA, srcB, dstB, rowsB, acc, semA, semB):
    cid = lax.axis_index("c")
    sid = lax.axis_index("s")
    r0 = sid * RZ
    for j in range(RZ // K):
        pltpu.sync_copy(z_hbm, acc.at[pl.ds(r0 + j * K, K)])
    plsc.subcore_barrier()

    def it(i, carry):
        c = (sid * NC + cid) + NW * i
        live = c < NCHUNK

        @pl.when(live & (cid == 0))
        def _():
            pltpu.sync_copy(src_hbm.at[pl.ds(c * K, K)], srcA)
            pltpu.sync_copy(dst_hbm.at[pl.ds(c * K, K)], dstA)
            pltpu.async_copy(h_hbm.at[srcA], rowsA, semA).wait()

        @pl.when(live & (cid == 1))
        def _():
            pltpu.sync_copy(src_hbm.at[pl.ds(c * K, K)], srcB)
            pltpu.sync_copy(dst_hbm.at[pl.ds(c * K, K)], dstB)
            pltpu.async_copy(h_hbm.at[srcB], rowsB, semB).wait()

        for t in range(NS):
            plsc.subcore_barrier()

            @pl.when(live & (sid == t) & (cid == 0))
            def _():
                pltpu.sync_copy(rowsA, acc.at[dstA], add=True)

            @pl.when(live & (sid == t) & (cid == 1))
            def _():
                pltpu.sync_copy(rowsB, acc.at[dstB], add=True)

        return carry

    lax.fori_loop(0, ITERS, it, 0)
    plsc.subcore_barrier()
    pltpu.sync_copy(acc.at[pl.ds(r0, RZ)],
                    out_hbm.at[pl.ds(cid * NPAD + r0, RZ)])


_sc_agg = pl.kernel(
    _sc_agg_body,
    out_type=jax.ShapeDtypeStruct((NC * NPAD, D), jnp.float32),
    mesh=_MESH,
    scratch_types=[
        pltpu.VMEM((K,), jnp.int32),
        pltpu.VMEM((K,), jnp.int32),
        pltpu.VMEM((K, D), jnp.float32),
        pltpu.VMEM((K,), jnp.int32),
        pltpu.VMEM((K,), jnp.int32),
        pltpu.VMEM((K, D), jnp.float32),
        pltpu.VMEM_SHARED((NPAD, D), jnp.float32),
        pltpu.SemaphoreType.DMA,
        pltpu.SemaphoreType.DMA,
    ],
)


def _sc_deg_body(ones_hbm, dst_hbm, z_hbm, out_hbm,
                 dstA, dstB, rows_v, acc, sem):
    cid = lax.axis_index("c")
    sid = lax.axis_index("s")
    r0 = sid * RZ
    pltpu.sync_copy(ones_hbm, rows_v)
    for j in range(RZ // KD):
        pltpu.sync_copy(z_hbm, acc.at[pl.ds(r0 + j * KD, KD)])
    plsc.subcore_barrier()

    def it(i, carry):
        c = (sid * NC + cid) + NW * i
        live = c < NCHUNKD

        @pl.when(live & (cid == 0))
        def _():
            pltpu.sync_copy(dst_hbm.at[pl.ds(c * KD, KD)], dstA)

        @pl.when(live & (cid == 1))
        def _():
            pltpu.sync_copy(dst_hbm.at[pl.ds(c * KD, KD)], dstB)

        for t in range(NS):
            plsc.subcore_barrier()

            @pl.when(live & (sid == t) & (cid == 0))
            def _():
                pltpu.sync_copy(rows_v, acc.at[dstA], add=True)

            @pl.when(live & (sid == t) & (cid == 1))
            def _():
                pltpu.sync_copy(rows_v, acc.at[dstB], add=True)

        return carry

    lax.fori_loop(0, ITERSD, it, 0)
    plsc.subcore_barrier()
    pltpu.sync_copy(acc.at[pl.ds(r0, RZ)],
                    out_hbm.at[pl.ds(cid * NPAD + r0, RZ)])


_sc_deg = pl.kernel(
    _sc_deg_body,
    out_type=jax.ShapeDtypeStruct((NC * NPAD, D), jnp.float32),
    mesh=_MESH,
    scratch_types=[
        pltpu.VMEM((KD,), jnp.int32),
        pltpu.VMEM((KD,), jnp.int32),
        pltpu.VMEM((KD, D), jnp.float32),
        pltpu.VMEM_SHARED((NPAD, D), jnp.float32),
        pltpu.SemaphoreType.DMA,
    ],
)


def _tc_pre_body(x_ref, w_ref, aggd_ref, h1p_ref, dis_ref):
    a = aggd_ref[...]
    deg = a[:N, 0:1] + a[NPAD:NPAD + N, 0:1] + 1.0
    dis = lax.rsqrt(deg)
    h = jnp.dot(x_ref[...], w_ref[...], preferred_element_type=jnp.float32)
    h1p_ref[...] = h * dis
    dis_ref[...] = dis


_tc_pre = pl.pallas_call(
    _tc_pre_body,
    out_shape=(
        jax.ShapeDtypeStruct((N, D), jnp.float32),
        jax.ShapeDtypeStruct((N, 1), jnp.float32),
    ),
)


def _tc_mid_body(p_ref, h1p_ref, dis_ref, b1_ref, w2_ref, h2p_ref):
    p = p_ref[...]
    dis = dis_ref[...]
    a = dis * (p[:N] + p[NPAD:NPAD + N] + h1p_ref[...]) + b1_ref[...]
    z = jnp.maximum(a, 0.0)
    h2p_ref[...] = dis * jnp.dot(z, w2_ref[...],
                                 preferred_element_type=jnp.float32)


_tc_mid = pl.pallas_call(
    _tc_mid_body,
    out_shape=jax.ShapeDtypeStruct((N, D), jnp.float32),
)


def _tc_post_body(p_ref, h2p_ref, dis_ref, b2_ref, out_ref):
    p = p_ref[...]
    out_ref[...] = dis_ref[...] * (p[:N] + p[NPAD:NPAD + N]
                                   + h2p_ref[...]) + b2_ref[...]


_tc_post = pl.pallas_call(
    _tc_post_body,
    out_shape=jax.ShapeDtypeStruct((N, D), jnp.float32),
)


def kernel(x, edge_index, W1, b1, W2, b2):
    src = edge_index[0]
    dst = edge_index[1]
    z128 = jnp.zeros((K, D), jnp.float32)
    zd = jnp.zeros((KD, D), jnp.float32)
    ones_kd = jnp.ones((KD, D), jnp.float32)

    aggd = _sc_deg(ones_kd, dst, zd)
    h1p, dis = _tc_pre(x, W1, aggd)
    p1 = _sc_agg(h1p, src, dst, z128)
    h2p = _tc_mid(p1, h1p, dis, b1.reshape(1, D), W2)
    p2 = _sc_agg(h2p, src, dst, z128)
    return _tc_post(p2, h2p, dis, b2.reshape(1, D))
